# baseline (speedup 1.0000x reference)
import functools

import jax
import jax.numpy as jnp
from jax import lax
from jax.experimental import pallas as pl
from jax.experimental.pallas import tpu as pltpu
from jax.experimental.pallas import tpu_sc as plsc


@functools.lru_cache(maxsize=None)
def _make_gather(V, D, B):
    info = plsc.get_sparse_core_info()
    NC, NS = info.num_cores, info.num_subcores
    NW = NC * NS
    assert B % NW == 0
    bpw = B // NW
    mesh = plsc.VectorSubcoreMesh(core_axis_name="c", subcore_axis_name="s")

    @functools.partial(
        pl.kernel,
        mesh=mesh,
        compiler_params=pltpu.CompilerParams(use_tc_tiling_on_sc=False),
        out_type=[
            jax.ShapeDtypeStruct((B, D), jnp.float32),
            jax.ShapeDtypeStruct((B,), jnp.float32),
        ],
        scratch_types=[
            pltpu.VMEM((bpw,), jnp.int32),
            pltpu.VMEM((bpw, D), jnp.float32),
            pltpu.VMEM((bpw,), jnp.float32),
            pltpu.SemaphoreType.DMA,
        ],
    )
    def gather(ids_hbm, emb_hbm, biastab_hbm, emb_out, bias_out,
               idx_v, rows_v, bias_v, sem):
        wid = lax.axis_index("s") * NC + lax.axis_index("c")
        base = wid * bpw
        pltpu.sync_copy(ids_hbm.at[pl.ds(base, bpw)], idx_v)
        cp_rows = pltpu.async_copy(emb_hbm.at[idx_v], rows_v, sem)
        cp_bias = pltpu.async_copy(biastab_hbm.at[idx_v], bias_v, sem)
        cp_rows.wait()
        cp_bias.wait()
        pltpu.sync_copy(rows_v, emb_out.at[pl.ds(base, bpw)])
        pltpu.sync_copy(bias_v, bias_out.at[pl.ds(base, bpw)])

    return gather


def _combine_body(x_ref, w1_ref, b1_ref, w2_ref, b2_ref, w3_ref, b3_ref,
                  emb_ref, bias_ref, out_ref, wc_ref, bc_ref):
    @pl.when(pl.program_id(0) == 0)
    def _():
        w12 = jnp.dot(w1_ref[...], w2_ref[...],
                      preferred_element_type=jnp.float32)
        wc_ref[...] = jnp.dot(w12, w3_ref[...],
                              preferred_element_type=jnp.float32)
        t = jnp.dot(b1_ref[...], w2_ref[...],
                    preferred_element_type=jnp.float32) + b2_ref[...]
        bc_ref[...] = jnp.dot(t, w3_ref[...],
                              preferred_element_type=jnp.float32) + b3_ref[...]

    rf = jnp.dot(x_ref[...], wc_ref[...],
                 preferred_element_type=jnp.float32) + bc_ref[...]
    s = jnp.sum(rf * emb_ref[...], axis=1, keepdims=True) + bias_ref[...]
    out_ref[...] = jax.nn.sigmoid(s)


@functools.lru_cache(maxsize=None)
def _make_combine(B, F, H1, H2, D, BLK):
    grid = (B // BLK,)
    return pl.pallas_call(
        _combine_body,
        grid=grid,
        in_specs=[
            pl.BlockSpec((BLK, F), lambda i: (i, 0)),
            pl.BlockSpec((F, H1), lambda i: (0, 0)),
            pl.BlockSpec((1, H1), lambda i: (0, 0)),
            pl.BlockSpec((H1, H2), lambda i: (0, 0)),
            pl.BlockSpec((1, H2), lambda i: (0, 0)),
            pl.BlockSpec((H2, D), lambda i: (0, 0)),
            pl.BlockSpec((1, D), lambda i: (0, 0)),
            pl.BlockSpec((BLK, D), lambda i: (i, 0)),
            pl.BlockSpec((BLK, 1), lambda i: (i, 0)),
        ],
        out_specs=pl.BlockSpec((BLK, 1), lambda i: (i, 0)),
        out_shape=jax.ShapeDtypeStruct((B, 1), jnp.float32),
        scratch_shapes=[
            pltpu.VMEM((F, D), jnp.float32),
            pltpu.VMEM((1, D), jnp.float32),
        ],
    )


def kernel(user_ids, restaurant_features, user_emb_table, user_bias_table,
           W1, b1, W2, b2, W3, b3):
    B, F = restaurant_features.shape
    V, D = user_emb_table.shape
    H1 = W1.shape[1]
    H2 = W2.shape[1]

    ids = user_ids.reshape(B).astype(jnp.int32)
    bias_tab = user_bias_table.reshape(V)
    emb, bias = _make_gather(V, D, B)(ids, user_emb_table, bias_tab)

    out = _make_combine(B, F, H1, H2, D, 2048)(
        restaurant_features, W1, b1.reshape(1, H1), W2, b2.reshape(1, H2),
        W3, b3.reshape(1, D), emb, bias.reshape(B, 1))
    return out
